# baseline (speedup 1.0000x reference)
import jax
import jax.numpy as jnp
from jax import lax
from jax.experimental import pallas as pl
from jax.experimental.pallas import tpu as pltpu
from jax.experimental.pallas import tpu_sc as plsc

NUM_EXPERTS = 64
NC, NS, L = 2, 16, 16
NW = NC * NS


def _sc_body(mask_hbm, imp_hbm, buf, impbuf):
    wid = lax.axis_index("s") * NC + lax.axis_index("c")
    per = buf.shape[0]
    n_own = (NUM_EXPERTS // NW)
    e0 = n_own * wid
    lane = lax.iota(jnp.int32, L)
    one = jnp.full((L,), 1.0, dtype=buf.dtype)
    zero = jnp.zeros((L,), dtype=buf.dtype)

    for k in range(n_own):
        vecs = [
            jnp.where(lane + (j * L) == e0 + k, one, zero)
            for j in range(NUM_EXPERTS // L)
        ]

        def fill(r, carry, vecs=vecs):
            for j in range(NUM_EXPERTS // L):
                buf[r, pl.ds(j * L, L)] = vecs[j]
            return carry

        lax.fori_loop(0, per, fill, 0)
        pltpu.sync_copy(buf, mask_hbm.at[pl.ds((e0 + k) * per, per)])

    @pl.when(wid % (NW // (NUM_EXPERTS // L)) == 0)
    def _():
        impbuf[...] = jnp.full((L,), float(per), dtype=impbuf.dtype)
        chunk = wid // (NW // (NUM_EXPERTS // L))
        pltpu.sync_copy(impbuf, imp_hbm.at[pl.ds(chunk * L, L)])


def kernel(x):
    n = x.shape[0]
    assert n % (NW * 8) == 0 and (n // NUM_EXPERTS) % 2 == 0
    dt = x.dtype
    mesh = plsc.VectorSubcoreMesh(core_axis_name="c", subcore_axis_name="s")
    f = pl.kernel(
        _sc_body,
        out_type=(
            jax.ShapeDtypeStruct((n, NUM_EXPERTS), dt),
            jax.ShapeDtypeStruct((NUM_EXPERTS,), dt),
        ),
        mesh=mesh,
        scratch_types=[
            pltpu.VMEM((n // NUM_EXPERTS, NUM_EXPERTS), dt),
            pltpu.VMEM((L,), dt),
        ],
    )
    mask, imp = f()
    return (mask, mask, imp, imp)


# speedup vs baseline: 1.0695x; 1.0695x over previous
import jax
import jax.numpy as jnp
from jax import lax
from jax.experimental import pallas as pl
from jax.experimental.pallas import tpu as pltpu
from jax.experimental.pallas import tpu_sc as plsc

NUM_EXPERTS = 64
NC, NS, L = 2, 16, 16
NW = NC * NS
CROW = 128
UNROLL = 8


def _sc_body(mask_hbm, imp_hbm, buf0, buf1, impbuf, sem):
    wid = lax.axis_index("s") * NC + lax.axis_index("c")
    per = mask_hbm.shape[0] // NUM_EXPERTS
    n_own = NUM_EXPERTS // NW
    e0 = n_own * wid
    lane = lax.iota(jnp.int32, L)
    one = jnp.full((L,), 1.0, dtype=buf0.dtype)
    zero = jnp.zeros((L,), dtype=buf0.dtype)
    bufs = [buf0, buf1]

    for k in range(n_own):
        vecs = [
            jnp.where(lane + (j * L) == e0 + k, one, zero)
            for j in range(NUM_EXPERTS // L)
        ]

        def fill(g, carry, vecs=vecs, buf=bufs[k]):
            for u in range(UNROLL):
                for j in range(NUM_EXPERTS // L):
                    buf[g * UNROLL + u, pl.ds(j * L, L)] = vecs[j]
            return carry

        lax.fori_loop(0, CROW // UNROLL, fill, 0)

    descs = []
    for k in range(n_own):
        for c in range(per // CROW):
            d = pltpu.make_async_copy(
                bufs[k],
                mask_hbm.at[pl.ds((e0 + k) * per + c * CROW, CROW)],
                sem,
            )
            d.start()
            descs.append(d)
    for d in descs:
        d.wait()

    @pl.when(wid % (NW // (NUM_EXPERTS // L)) == 0)
    def _():
        impbuf[...] = jnp.full((L,), float(per), dtype=impbuf.dtype)
        chunk = wid // (NW // (NUM_EXPERTS // L))
        pltpu.sync_copy(impbuf, imp_hbm.at[pl.ds(chunk * L, L)])


def kernel(x):
    n = x.shape[0]
    assert n % NUM_EXPERTS == 0 and (n // NUM_EXPERTS) % CROW == 0
    dt = x.dtype
    mesh = plsc.VectorSubcoreMesh(core_axis_name="c", subcore_axis_name="s")
    f = pl.kernel(
        _sc_body,
        out_type=(
            jax.ShapeDtypeStruct((n, NUM_EXPERTS), dt),
            jax.ShapeDtypeStruct((NUM_EXPERTS,), dt),
        ),
        mesh=mesh,
        scratch_types=[
            pltpu.VMEM((CROW, NUM_EXPERTS), dt),
            pltpu.VMEM((CROW, NUM_EXPERTS), dt),
            pltpu.VMEM((L,), dt),
            pltpu.SemaphoreType.DMA,
        ],
    )
    mask, imp = f()
    return (mask, mask, imp, imp)


# speedup vs baseline: 1.2993x; 1.2149x over previous
import jax
import jax.numpy as jnp
from jax.experimental import pallas as pl
from jax.experimental.pallas import tpu as pltpu

NUM_EXPERTS = 64
NCHUNKS = 16


def _body(mask_hbm, routes_hbm, imp_ref, load_ref, buf, sems):
    n = buf.shape[0]
    per = n // NUM_EXPERTS
    ch_rows = n // NCHUNKS
    epc = NUM_EXPERTS // NCHUNKS
    col = jax.lax.broadcasted_iota(jnp.int32, (per, NUM_EXPERTS), 1)
    acc = jnp.zeros((NUM_EXPERTS,), imp_ref.dtype)
    for c in range(NCHUNKS):

        def fill(k, a, c=c):
            e = c * epc + k
            pat = (col == e).astype(buf.dtype)
            buf[pl.ds(c * ch_rows + k * per, per), :] = pat
            return a + jnp.sum(pat, axis=0)

        acc = jax.lax.fori_loop(0, epc, fill, acc)
        for t, dst in enumerate((mask_hbm, routes_hbm)):
            pltpu.make_async_copy(
                buf.at[pl.ds(c * ch_rows, ch_rows), :],
                dst.at[pl.ds(c * ch_rows, ch_rows), :],
                sems.at[2 * c + t],
            ).start()
    for c in range(NCHUNKS):
        for t, dst in enumerate((mask_hbm, routes_hbm)):
            pltpu.make_async_copy(
                buf.at[pl.ds(c * ch_rows, ch_rows), :],
                dst.at[pl.ds(c * ch_rows, ch_rows), :],
                sems.at[2 * c + t],
            ).wait()
    imp_ref[...] = acc
    load_ref[...] = acc


def kernel(x):
    n = x.shape[0]
    assert n % (NCHUNKS * NUM_EXPERTS) == 0
    dt = x.dtype
    mask, routes, imp, load = pl.pallas_call(
        _body,
        out_shape=(
            jax.ShapeDtypeStruct((n, NUM_EXPERTS), dt),
            jax.ShapeDtypeStruct((n, NUM_EXPERTS), dt),
            jax.ShapeDtypeStruct((NUM_EXPERTS,), dt),
            jax.ShapeDtypeStruct((NUM_EXPERTS,), dt),
        ),
        out_specs=(
            pl.BlockSpec(memory_space=pltpu.MemorySpace.HBM),
            pl.BlockSpec(memory_space=pltpu.MemorySpace.HBM),
            pl.BlockSpec(memory_space=pltpu.MemorySpace.VMEM),
            pl.BlockSpec(memory_space=pltpu.MemorySpace.VMEM),
        ),
        scratch_shapes=[
            pltpu.VMEM((n, NUM_EXPERTS), dt),
            pltpu.SemaphoreType.DMA((2 * NCHUNKS,)),
        ],
    )()
    return (mask, routes, imp, load)


# speedup vs baseline: 6.3575x; 4.8929x over previous
import jax
import jax.numpy as jnp
from jax.experimental import pallas as pl
from jax.experimental.pallas import tpu as pltpu

NUM_EXPERTS = 64
BANDS = 8


def _body(maskT_hbm, routesT_hbm, imp_ref, load_ref, buf, sems):
    n_exp, n = buf.shape
    per = n // n_exp
    bre = n_exp // BANDS
    for b in range(BANDS):
        r = jax.lax.broadcasted_iota(jnp.int32, (bre, n), 0)
        c = jax.lax.broadcasted_iota(jnp.int32, (bre, n), 1)
        low = (r + b * bre) * per
        pat = ((c >= low) & (c < low + per)).astype(buf.dtype)
        buf[pl.ds(b * bre, bre), :] = pat
        s = jnp.sum(pat, axis=1)
        imp_ref[pl.ds(b * bre, bre)] = s
        load_ref[pl.ds(b * bre, bre)] = s
        for t, dst in enumerate((maskT_hbm, routesT_hbm)):
            pltpu.make_async_copy(
                buf.at[pl.ds(b * bre, bre), :],
                dst.at[pl.ds(b * bre, bre), :],
                sems.at[2 * b + t],
            ).start()
    for b in range(BANDS):
        for t, dst in enumerate((maskT_hbm, routesT_hbm)):
            pltpu.make_async_copy(
                buf.at[pl.ds(b * bre, bre), :],
                dst.at[pl.ds(b * bre, bre), :],
                sems.at[2 * b + t],
            ).wait()


def kernel(x):
    n = x.shape[0]
    assert n % NUM_EXPERTS == 0 and NUM_EXPERTS % BANDS == 0
    dt = x.dtype
    maskT, routesT, imp, load = pl.pallas_call(
        _body,
        out_shape=(
            jax.ShapeDtypeStruct((NUM_EXPERTS, n), dt),
            jax.ShapeDtypeStruct((NUM_EXPERTS, n), dt),
            jax.ShapeDtypeStruct((NUM_EXPERTS,), dt),
            jax.ShapeDtypeStruct((NUM_EXPERTS,), dt),
        ),
        out_specs=(
            pl.BlockSpec(memory_space=pltpu.MemorySpace.HBM),
            pl.BlockSpec(memory_space=pltpu.MemorySpace.HBM),
            pl.BlockSpec(memory_space=pltpu.MemorySpace.VMEM),
            pl.BlockSpec(memory_space=pltpu.MemorySpace.VMEM),
        ),
        scratch_shapes=[
            pltpu.VMEM((NUM_EXPERTS, n), dt),
            pltpu.SemaphoreType.DMA((2 * BANDS,)),
        ],
    )()
    return (maskT.T, routesT.T, imp, load)
